# Initial kernel scaffold; baseline (speedup 1.0000x reference)
import jax
import jax.numpy as jnp
from jax.experimental import pallas as pl


def kernel(recommendations, distance_matrix):
    raise NotImplementedError("write your pallas kernel here")


# speedup vs baseline: 1.7388x; 1.7388x over previous
import functools

import numpy as np
import jax
import jax.numpy as jnp
from jax import lax
from jax.experimental import pallas as pl
from jax.experimental.pallas import tpu as pltpu
from jax.experimental.pallas import tpu_sc as plsc

_B, _K, _V = 1024, 20, 1000
_NC, _NS, _L = 2, 16, 16
_NW = _NC * _NS
_UPW = _B // _NW
_NPAIR = _K * (_K - 1) // 2
_PPAD = 192
_PW = _UPW * _PPAD
_ROWS = _PW // 128


def _patterns():
    a_pos, c_pos, w01 = [], [], []
    for a in range(_K):
        for c in range(a + 1, _K):
            a_pos.append(a)
            c_pos.append(c)
            w01.append(1.0)
    while len(a_pos) < _PPAD:
        a_pos.append(0)
        c_pos.append(0)
        w01.append(0.0)
    a_pos = np.asarray(a_pos, np.int32)
    c_pos = np.asarray(c_pos, np.int32)
    w01 = np.asarray(w01, np.float32)
    u = np.arange(_UPW, dtype=np.int32)[:, None] * _K
    pa = (u + a_pos[None, :]).reshape(-1)
    pc = (u + c_pos[None, :]).reshape(-1)
    scale = np.float32(1.0 / (_B * _K * (_K - 1)))
    w = np.broadcast_to(w01[None, :] * scale, (_UPW, _PPAD)).reshape(-1).copy()
    return pa, pc, w


_PA_NP, _PC_NP, _W_NP = _patterns()


def _make_sc_kernel():
    mesh = plsc.VectorSubcoreMesh(core_axis_name="c", subcore_axis_name="s")

    @functools.partial(
        pl.kernel,
        mesh=mesh,
        compiler_params=pltpu.CompilerParams(needs_layout_passes=False),
        out_type=jax.ShapeDtypeStruct((_NW * _L,), jnp.float32),
        scratch_types=[
            pltpu.VMEM((_UPW * _K,), jnp.int32),
            pltpu.VMEM((_PW,), jnp.int32),
            pltpu.VMEM((_PW,), jnp.int32),
            pltpu.VMEM((_PW,), jnp.float32),
            pltpu.VMEM((_ROWS, 128), jnp.int32),
            pltpu.VMEM((_ROWS, 128), jnp.float32),
            pltpu.VMEM((_L,), jnp.float32),
            pltpu.SemaphoreType.DMA,
        ],
    )
    def ild_kernel(rec_hbm, d_hbm, pa_hbm, pc_hbm, w_hbm, out_hbm,
                   rec_v, pa_v, pc_v, w_v, idx_v, val_v, acc_v, sem):
        wid = lax.axis_index("s") * _NC + lax.axis_index("c")
        pltpu.sync_copy(rec_hbm.at[pl.ds(wid * (_UPW * _K), _UPW * _K)], rec_v)
        pltpu.sync_copy(pa_hbm, pa_v)
        pltpu.sync_copy(pc_hbm, pc_v)
        pltpu.sync_copy(w_hbm, w_v)

        def build(j, carry):
            for t in range(8):
                sl = pl.ds(j * 128 + t * 16, _L)
                ia = plsc.load_gather(rec_v, [pa_v[sl]])
                ic = plsc.load_gather(rec_v, [pc_v[sl]])
                idx_v[j, pl.ds(t * 16, _L)] = ia * _V + ic
            return carry

        lax.fori_loop(0, _ROWS, build, 0)

        copies = [
            pltpu.async_copy(d_hbm.at[idx_v.at[j]], val_v.at[j], sem)
            for j in range(_ROWS)
        ]
        for cp in copies:
            cp.wait()

        def red(j, acc):
            for t in range(8):
                acc = acc + (val_v[j, pl.ds(t * 16, _L)]
                             * w_v[pl.ds(j * 128 + t * 16, _L)])
            return acc

        acc = lax.fori_loop(0, _ROWS, red, jnp.zeros((_L,), jnp.float32))
        acc_v[...] = acc
        pltpu.sync_copy(acc_v, out_hbm.at[pl.ds(wid * _L, _L)])

    return ild_kernel


_SC_KERNEL = _make_sc_kernel()


def kernel(recommendations, distance_matrix):
    rec = recommendations.astype(jnp.int32).reshape(-1)
    dflat = distance_matrix.reshape(-1)
    pa = jnp.asarray(_PA_NP)
    pc = jnp.asarray(_PC_NP)
    w = jnp.asarray(_W_NP)
    partials = _SC_KERNEL(rec, dflat, pa, pc, w)
    return jnp.sum(partials)


# speedup vs baseline: 1.9459x; 1.1191x over previous
import functools

import numpy as np
import jax
import jax.numpy as jnp
from jax import lax
from jax.experimental import pallas as pl
from jax.experimental.pallas import tpu as pltpu
from jax.experimental.pallas import tpu_sc as plsc

_B, _K, _V = 1024, 20, 1000
_NC, _NS, _L = 2, 16, 16
_NW = _NC * _NS
_UPW = _B // _NW
_NPAIR = _K * (_K - 1) // 2
_PPAD = 192
_CPU = _PPAD // _L
_PW = _UPW * _PPAD
_ROWS = _PW // 128
_HROWS = _ROWS // 2


def _patterns():
    a_pos, c_pos, w = [], [], []
    for a in range(_K):
        for c in range(a + 1, _K):
            a_pos.append(a)
            c_pos.append(c)
            w.append(1.0 / (_B * _K * (_K - 1)))
    while len(a_pos) < _PPAD:
        a_pos.append(0)
        c_pos.append(0)
        w.append(0.0)
    return (np.asarray(a_pos, np.int32), np.asarray(c_pos, np.int32),
            np.asarray(w, np.float32))


_PA_NP, _PC_NP, _W_NP = _patterns()


def _make_sc_kernel():
    mesh = plsc.VectorSubcoreMesh(core_axis_name="c", subcore_axis_name="s")

    @functools.partial(
        pl.kernel,
        mesh=mesh,
        compiler_params=pltpu.CompilerParams(needs_layout_passes=False),
        out_type=jax.ShapeDtypeStruct((_NW * _L,), jnp.float32),
        scratch_types=[
            pltpu.VMEM((_UPW * _K,), jnp.int32),
            pltpu.VMEM((_PPAD,), jnp.int32),
            pltpu.VMEM((_PPAD,), jnp.int32),
            pltpu.VMEM((_PPAD,), jnp.float32),
            pltpu.VMEM((_ROWS, 128), jnp.int32),
            pltpu.VMEM((_ROWS, 128), jnp.float32),
            pltpu.VMEM((_L,), jnp.float32),
            pltpu.SemaphoreType.DMA,
            pltpu.SemaphoreType.DMA,
        ],
    )
    def ild_kernel(rec_hbm, d_hbm, pa_hbm, pc_hbm, w_hbm, out_hbm,
                   rec_v, pa_v, pc_v, w_v, idx_v, val_v, acc_v,
                   sem_a, sem_b):
        wid = lax.axis_index("s") * _NC + lax.axis_index("c")
        pltpu.sync_copy(rec_hbm.at[pl.ds(wid * (_UPW * _K), _UPW * _K)], rec_v)
        pltpu.sync_copy(pa_hbm, pa_v)
        pltpu.sync_copy(pc_hbm, pc_v)
        pltpu.sync_copy(w_hbm, w_v)

        def build_fire(j, sem):
            for t in range(8):
                g = j * 8 + t
                u = g // _CPU
                ts = g % _CPU
                off = u * _K
                pa = pa_v[pl.ds(ts * _L, _L)] + off
                pc = pc_v[pl.ds(ts * _L, _L)] + off
                ia = plsc.load_gather(rec_v, [pa])
                ic = plsc.load_gather(rec_v, [pc])
                idx_v[j, pl.ds(t * _L, _L)] = ia * _V + ic
            pltpu.make_async_copy(d_hbm.at[idx_v.at[j]], val_v.at[j],
                                  sem).start()

        lax.fori_loop(0, _HROWS, lambda j, c: (build_fire(j, sem_a), c)[1], 0)
        lax.fori_loop(_HROWS, _ROWS,
                      lambda j, c: (build_fire(j, sem_b), c)[1], 0)

        def drain(sem, lo, hi):
            for j in range(lo, hi):
                pltpu.make_async_copy(d_hbm.at[pl.ds(0, 128)], val_v.at[j],
                                      sem).wait()

        def red(j, acc):
            for t in range(8):
                ts = (j * 8 + t) % _CPU
                acc = acc + (val_v[j, pl.ds(t * _L, _L)]
                             * w_v[pl.ds(ts * _L, _L)])
            return acc

        drain(sem_a, 0, _HROWS)
        acc = lax.fori_loop(0, _HROWS, red, jnp.zeros((_L,), jnp.float32))
        drain(sem_b, _HROWS, _ROWS)
        acc = lax.fori_loop(_HROWS, _ROWS, red, acc)
        acc_v[...] = acc
        pltpu.sync_copy(acc_v, out_hbm.at[pl.ds(wid * _L, _L)])

    return ild_kernel


_SC_KERNEL = _make_sc_kernel()


def kernel(recommendations, distance_matrix):
    rec = recommendations.astype(jnp.int32).reshape(-1)
    dflat = distance_matrix.reshape(-1)
    pa = jnp.asarray(_PA_NP)
    pc = jnp.asarray(_PC_NP)
    w = jnp.asarray(_W_NP)
    partials = _SC_KERNEL(rec, dflat, pa, pc, w)
    return jnp.sum(partials)


# speedup vs baseline: 1.9977x; 1.0266x over previous
import functools

import numpy as np
import jax
import jax.numpy as jnp
from jax import lax
from jax.experimental import pallas as pl
from jax.experimental.pallas import tpu as pltpu
from jax.experimental.pallas import tpu_sc as plsc

_B, _K, _V = 1024, 20, 1000
_NC, _NS, _L = 2, 16, 16
_NW = _NC * _NS
_UPW = _B // _NW
_NPAIR = _K * (_K - 1) // 2
_PPAD = 192
_CPU = _PPAD // _L
_PW = _UPW * _PPAD
_ROWS = _PW // 128
_HROWS = _ROWS // 2
_SCALE = np.float32(1.0 / (_B * _K * (_K - 1)))


def _patterns():
    a_pos, c_pos = [], []
    for a in range(_K):
        for c in range(a + 1, _K):
            a_pos.append(a)
            c_pos.append(c)
    while len(a_pos) < _PPAD:
        a_pos.append(0)
        c_pos.append(0)
    return np.asarray(a_pos, np.int32), np.asarray(c_pos, np.int32)


_PA_NP, _PC_NP = _patterns()


def _make_sc_kernel():
    mesh = plsc.VectorSubcoreMesh(core_axis_name="c", subcore_axis_name="s")

    @functools.partial(
        pl.kernel,
        mesh=mesh,
        compiler_params=pltpu.CompilerParams(needs_layout_passes=False),
        out_type=jax.ShapeDtypeStruct((_NW * _L,), jnp.float32),
        scratch_types=[
            pltpu.VMEM((_UPW, _K), jnp.int32),
            pltpu.VMEM((_PPAD,), jnp.int32),
            pltpu.VMEM((_PPAD,), jnp.int32),
            pltpu.VMEM((_ROWS, 128), jnp.int32),
            pltpu.VMEM((_ROWS, 128), jnp.float32),
            pltpu.VMEM((_L,), jnp.float32),
            pltpu.SemaphoreType.DMA,
            pltpu.SemaphoreType.DMA,
        ],
    )
    def ild_kernel(rec_hbm, d_hbm, pa_hbm, pc_hbm, out_hbm,
                   rec_v, pa_v, pc_v, idx_v, val_v, acc_v,
                   sem_a, sem_b):
        wid = lax.axis_index("s") * _NC + lax.axis_index("c")
        pltpu.sync_copy(rec_hbm.at[pl.ds(wid * _UPW, _UPW), :], rec_v)
        pltpu.sync_copy(pa_hbm, pa_v)
        pltpu.sync_copy(pc_hbm, pc_v)

        def build_fire(j, sem):
            for t in range(8):
                g = j * 8 + t
                u = g // _CPU
                ts = g % _CPU
                uvec = jnp.full((_L,), u, jnp.int32)
                pa = pa_v[pl.ds(ts * _L, _L)]
                pc = pc_v[pl.ds(ts * _L, _L)]
                ia = plsc.load_gather(rec_v, [uvec, pa])
                ic = plsc.load_gather(rec_v, [uvec, pc])
                idx_v[j, pl.ds(t * _L, _L)] = ia * _V + ic
            pltpu.make_async_copy(d_hbm.at[idx_v.at[j]], val_v.at[j],
                                  sem).start()

        lax.fori_loop(0, _HROWS, lambda j, c: (build_fire(j, sem_a), c)[1], 0)
        lax.fori_loop(_HROWS, _ROWS,
                      lambda j, c: (build_fire(j, sem_b), c)[1], 0)

        def drain(sem, lo, hi):
            for j in range(lo, hi):
                pltpu.make_async_copy(d_hbm.at[pl.ds(0, 128)], val_v.at[j],
                                      sem).wait()

        lane = lax.iota(jnp.int32, _L)

        def red(j, acc):
            for t in range(8):
                ts = (j * 8 + t) % _CPU
                w = jnp.where(ts * _L + lane < _NPAIR, _SCALE,
                              jnp.float32(0.0))
                acc = acc + val_v[j, pl.ds(t * _L, _L)] * w
            return acc

        drain(sem_a, 0, _HROWS)
        acc = lax.fori_loop(0, _HROWS, red, jnp.zeros((_L,), jnp.float32))
        drain(sem_b, _HROWS, _ROWS)
        acc = lax.fori_loop(_HROWS, _ROWS, red, acc)
        acc_v[...] = acc
        pltpu.sync_copy(acc_v, out_hbm.at[pl.ds(wid * _L, _L)])

    return ild_kernel


_SC_KERNEL = _make_sc_kernel()


def kernel(recommendations, distance_matrix):
    rec = recommendations.astype(jnp.int32)
    dflat = distance_matrix.reshape(-1)
    pa = jnp.asarray(_PA_NP)
    pc = jnp.asarray(_PC_NP)
    partials = _SC_KERNEL(rec, dflat, pa, pc)
    return jnp.sum(partials)


# speedup vs baseline: 2.1577x; 1.0801x over previous
import functools

import numpy as np
import jax
import jax.numpy as jnp
from jax import lax
from jax.experimental import pallas as pl
from jax.experimental.pallas import tpu as pltpu
from jax.experimental.pallas import tpu_sc as plsc

_B, _K, _V = 1024, 20, 1000
_NC, _NS, _L = 2, 16, 16
_NW = _NC * _NS
_UPW = _B // _NW
_NPAIR = _K * (_K - 1) // 2
_PPAD = 192
_CPU = _PPAD // _L
_RPU = 2
_RLEN = _PPAD // _RPU
_CPR = _RLEN // _L
_ROWS = _UPW * _RPU
_HROWS = _ROWS // 2
_SCALE = np.float32(1.0 / (_B * _K * (_K - 1)))


_THRESH = [a * _K - a * (a + 1) // 2 for a in range(1, _K)]


def _make_sc_kernel():
    mesh = plsc.VectorSubcoreMesh(core_axis_name="c", subcore_axis_name="s")

    @functools.partial(
        pl.kernel,
        mesh=mesh,
        compiler_params=pltpu.CompilerParams(needs_layout_passes=False),
        out_type=jax.ShapeDtypeStruct((_NW * _L,), jnp.float32),
        scratch_types=[
            pltpu.VMEM((_UPW * _K,), jnp.int32),
            pltpu.VMEM((_PPAD,), jnp.int32),
            pltpu.VMEM((_PPAD,), jnp.int32),
            pltpu.VMEM((_ROWS, _RLEN), jnp.int32),
            pltpu.VMEM((_ROWS, _RLEN), jnp.float32),
            pltpu.VMEM((_L,), jnp.float32),
            pltpu.SemaphoreType.DMA,
            pltpu.SemaphoreType.DMA,
        ],
    )
    def ild_kernel(rec_hbm, d_hbm, out_hbm,
                   rec_v, pa_v, pc_v, idx_v, val_v, acc_v, sem_a, sem_b):
        wid = lax.axis_index("s") * _NC + lax.axis_index("c")
        pltpu.sync_copy(rec_hbm.at[pl.ds(wid * (_UPW * _K), _UPW * _K)],
                        rec_v)

        lane = lax.iota(jnp.int32, _L)
        one = jnp.ones((_L,), jnp.int32)
        zero = jnp.zeros((_L,), jnp.int32)
        for t in range(_CPU):
            p = lane + t * _L
            a = zero
            for th in _THRESH:
                a = a + jnp.where(p >= th, one, zero)
            c = p - (a * _K - (a * (a + 1)) // 2) + a + 1
            c = jnp.minimum(c, _K - 1)
            pa_v[pl.ds(t * _L, _L)] = a
            pc_v[pl.ds(t * _L, _L)] = c

        def build_user(u, sem):
            base = u * _K
            for r in range(_RPU):
                for tc in range(_CPR):
                    t = r * _CPR + tc
                    ia = plsc.load_gather(rec_v, [pa_v[pl.ds(t * _L, _L)]
                                                  + base])
                    ic = plsc.load_gather(rec_v, [pc_v[pl.ds(t * _L, _L)]
                                                  + base])
                    idx_v[u * _RPU + r, pl.ds(tc * _L, _L)] = ia * _V + ic
                pltpu.make_async_copy(
                    d_hbm.at[idx_v.at[u * _RPU + r]],
                    val_v.at[u * _RPU + r], sem).start()

        lax.fori_loop(0, _UPW // 2,
                      lambda u, c: (build_user(u, sem_a), c)[1], 0)
        lax.fori_loop(_UPW // 2, _UPW,
                      lambda u, c: (build_user(u, sem_b), c)[1], 0)

        def drain(sem, lo, hi):
            for j in range(lo, hi):
                pltpu.make_async_copy(d_hbm.at[pl.ds(0, _RLEN)],
                                      val_v.at[j], sem).wait()

        tailmask = jnp.where(lane + (_CPU - 1) * _L < _NPAIR,
                             jnp.float32(1.0), jnp.float32(0.0))

        def red(j, acc):
            for tc in range(_CPR):
                v = val_v[j, pl.ds(tc * _L, _L)]
                if tc == _CPR - 1:
                    v = jnp.where((j % 2) == 1, v * tailmask, v)
                acc = acc + v
            return acc

        drain(sem_a, 0, _HROWS)
        acc = lax.fori_loop(0, _HROWS, red, jnp.zeros((_L,), jnp.float32))
        drain(sem_b, _HROWS, _ROWS)
        acc = lax.fori_loop(_HROWS, _ROWS, red, acc)
        acc_v[...] = acc * _SCALE
        pltpu.sync_copy(acc_v, out_hbm.at[pl.ds(wid * _L, _L)])

    return ild_kernel


_SC_KERNEL = _make_sc_kernel()


def kernel(recommendations, distance_matrix):
    rec = recommendations.astype(jnp.int32).reshape(-1)
    dflat = distance_matrix.reshape(-1)
    partials = _SC_KERNEL(rec, dflat)
    return jnp.sum(partials)
